# initial kernel scaffold (unmeasured)
import jax
import jax.numpy as jnp
from jax import lax
from jax.experimental import pallas as pl
from jax.experimental.pallas import tpu as pltpu

N_DEV = 4
B = 4
S_CHUNK = 512
K = 2048
N_OUT = 8192
NT = 1024
N_TILES = N_OUT // NT
TILES_PER_STEP = B * N_TILES
N_SLOTS = 4


def kernel(O, Wo):
    A = O.reshape(B, N_DEV * S_CHUNK, K)
    my = lax.axis_index("i")
    idxs = jnp.mod(my - 1 - jnp.arange(N_DEV, dtype=jnp.int32), N_DEV).astype(
        jnp.int32
    )

    def body(idx_ref, a_ref, w_ref, o_ref, send_slots, recvbuf, recv_vmem,
             send_sems, recv_sems, local_sem, exit_sem):
        h = pl.program_id(0)
        b = pl.program_id(1)
        n = pl.program_id(2)
        i = lax.axis_index("i")
        right = lax.rem(i + 1, N_DEV)
        left = lax.rem(i + N_DEV - 1, N_DEV)
        g = (h * B + b) * N_TILES + n
        slot = lax.rem(g, N_SLOTS)

        barrier = pltpu.get_barrier_semaphore()

        @pl.when(g == 0)
        def _entry_barrier():
            for nbr in (left, right):
                pl.semaphore_signal(
                    barrier, inc=1, device_id=(nbr,),
                    device_id_type=pl.DeviceIdType.MESH,
                )
            pl.semaphore_wait(barrier, 2)

        @pl.when((g >= N_SLOTS) & (g < 3 * TILES_PER_STEP + N_SLOTS))
        def _wait_old_send():
            d = pltpu.make_async_remote_copy(
                src_ref=send_slots.at[slot],
                dst_ref=recvbuf.at[0, 0, 0],
                send_sem=send_sems.at[slot],
                recv_sem=recv_sems.at[0, 0, 0],
                device_id=(right,),
                device_id_type=pl.DeviceIdType.MESH,
            )
            d.wait_send()

        @pl.when(h > 0)
        def _recv():
            rd = pltpu.make_async_remote_copy(
                src_ref=send_slots.at[0],
                dst_ref=recvbuf.at[h - 1, b, n],
                send_sem=send_sems.at[0],
                recv_sem=recv_sems.at[h - 1, b, n],
                device_id=(left,),
                device_id_type=pl.DeviceIdType.MESH,
            )
            rd.wait_recv()
            cp = pltpu.make_async_copy(
                recvbuf.at[h - 1, b, n], recv_vmem, local_sem
            )
            cp.start()
            cp.wait()

        @pl.when(h == 0)
        def _zero():
            recv_vmem[...] = jnp.zeros_like(recv_vmem)

        result = (
            jnp.dot(a_ref[0], w_ref[...], preferred_element_type=jnp.float32)
            + recv_vmem[...]
        )

        @pl.when(h < 3)
        def _send():
            send_slots[slot] = result
            rdma = pltpu.make_async_remote_copy(
                src_ref=send_slots.at[slot],
                dst_ref=recvbuf.at[h, b, n],
                send_sem=send_sems.at[slot],
                recv_sem=recv_sems.at[h, b, n],
                device_id=(right,),
                device_id_type=pl.DeviceIdType.MESH,
            )
            rdma.start()

        @pl.when(h == 3)
        def _store():
            o_ref[0] = result

        @pl.when(g == N_DEV * TILES_PER_STEP - 1)
        def _exit_barrier():
            for nbr in (left, right):
                pl.semaphore_signal(
                    exit_sem, inc=1, device_id=(nbr,),
                    device_id_type=pl.DeviceIdType.MESH,
                )
            pl.semaphore_wait(exit_sem, 2)

    grid_spec = pltpu.PrefetchScalarGridSpec(
        num_scalar_prefetch=1,
        grid=(N_DEV, B, N_TILES),
        in_specs=[
            pl.BlockSpec((1, S_CHUNK, K), lambda h, b, n, idx: (b, idx[h], 0)),
            pl.BlockSpec((K, NT), lambda h, b, n, idx: (0, n)),
        ],
        out_specs=pl.BlockSpec((1, S_CHUNK, NT), lambda h, b, n, idx: (b, 0, n)),
        scratch_shapes=[
            pltpu.VMEM((N_SLOTS, S_CHUNK, NT), jnp.float32),
            pl.ANY((3, B, N_TILES, S_CHUNK, NT), jnp.float32),
            pltpu.VMEM((S_CHUNK, NT), jnp.float32),
            pltpu.SemaphoreType.DMA((N_SLOTS,)),
            pltpu.SemaphoreType.DMA((3, B, N_TILES)),
            pltpu.SemaphoreType.DMA,
            pltpu.SemaphoreType.REGULAR,
        ],
    )

    return pl.pallas_call(
        body,
        grid_spec=grid_spec,
        out_shape=jax.ShapeDtypeStruct((B, S_CHUNK, N_OUT), jnp.float32),
        compiler_params=pltpu.CompilerParams(
            dimension_semantics=("arbitrary", "arbitrary", "arbitrary"),
            collective_id=0,
        ),
    )(idxs, A, Wo)


# baseline (device time: 2503168 ns/iter reference)
import jax
import jax.numpy as jnp
from jax import lax
from jax.experimental import pallas as pl
from jax.experimental.pallas import tpu as pltpu

N_DEV = 4
B = 4
S_CHUNK = 512
K = 2048
N_OUT = 8192
NT = 1024
N_TILES = N_OUT // NT
TILES_PER_STEP = B * N_TILES
N_SLOTS = 4


def kernel(O, Wo):
    A = O.reshape(B, N_DEV * S_CHUNK, K)
    my = lax.axis_index("i")
    idxs = jnp.mod(my - 1 - jnp.arange(N_DEV, dtype=jnp.int32), N_DEV).astype(
        jnp.int32
    )

    def body(idx_ref, a_ref, w_ref, o_ref, recvbuf, send_slots, recv_vmem,
             send_sems, recv_sems, local_sem, exit_sem):
        h = pl.program_id(0)
        b = pl.program_id(1)
        n = pl.program_id(2)
        i = lax.axis_index("i")
        right = lax.rem(i + 1, N_DEV)
        left = lax.rem(i + N_DEV - 1, N_DEV)
        g = (h * B + b) * N_TILES + n
        slot = lax.rem(g, N_SLOTS)

        barrier = pltpu.get_barrier_semaphore()

        @pl.when(g == 0)
        def _entry_barrier():
            for nbr in (left, right):
                pl.semaphore_signal(
                    barrier, inc=1, device_id=(nbr,),
                    device_id_type=pl.DeviceIdType.MESH,
                )
            pl.semaphore_wait(barrier, 2)

        @pl.when((g >= N_SLOTS) & (g < 3 * TILES_PER_STEP + N_SLOTS))
        def _wait_old_send():
            d = pltpu.make_async_remote_copy(
                src_ref=send_slots.at[slot],
                dst_ref=recvbuf.at[0, 0, 0],
                send_sem=send_sems.at[slot],
                recv_sem=recv_sems.at[0, 0, 0],
                device_id=(right,),
                device_id_type=pl.DeviceIdType.MESH,
            )
            d.wait_send()

        @pl.when(h > 0)
        def _recv():
            rd = pltpu.make_async_remote_copy(
                src_ref=send_slots.at[0],
                dst_ref=recvbuf.at[h - 1, b, n],
                send_sem=send_sems.at[0],
                recv_sem=recv_sems.at[h - 1, b, n],
                device_id=(left,),
                device_id_type=pl.DeviceIdType.MESH,
            )
            rd.wait_recv()
            cp = pltpu.make_async_copy(
                recvbuf.at[h - 1, b, n], recv_vmem, local_sem
            )
            cp.start()
            cp.wait()

        @pl.when(h == 0)
        def _zero():
            recv_vmem[...] = jnp.zeros_like(recv_vmem)

        result = (
            jnp.dot(a_ref[0], w_ref[...], preferred_element_type=jnp.float32)
            + recv_vmem[...]
        )

        @pl.when(h < 3)
        def _send():
            send_slots[slot] = result
            rdma = pltpu.make_async_remote_copy(
                src_ref=send_slots.at[slot],
                dst_ref=recvbuf.at[h, b, n],
                send_sem=send_sems.at[slot],
                recv_sem=recv_sems.at[h, b, n],
                device_id=(right,),
                device_id_type=pl.DeviceIdType.MESH,
            )
            rdma.start()

        @pl.when(h == 3)
        def _store():
            o_ref[0] = result

        @pl.when(g == N_DEV * TILES_PER_STEP - 1)
        def _exit_barrier():
            for nbr in (left, right):
                pl.semaphore_signal(
                    exit_sem, inc=1, device_id=(nbr,),
                    device_id_type=pl.DeviceIdType.MESH,
                )
            pl.semaphore_wait(exit_sem, 2)

    grid_spec = pltpu.PrefetchScalarGridSpec(
        num_scalar_prefetch=1,
        grid=(N_DEV, B, N_TILES),
        in_specs=[
            pl.BlockSpec((1, S_CHUNK, K), lambda h, b, n, idx: (b, idx[h], 0)),
            pl.BlockSpec((K, NT), lambda h, b, n, idx: (0, n)),
        ],
        out_specs=[
            pl.BlockSpec((1, S_CHUNK, NT), lambda h, b, n, idx: (b, 0, n)),
            pl.BlockSpec(memory_space=pl.ANY),
        ],
        scratch_shapes=[
            pltpu.VMEM((N_SLOTS, S_CHUNK, NT), jnp.float32),
            pltpu.VMEM((S_CHUNK, NT), jnp.float32),
            pltpu.SemaphoreType.DMA((N_SLOTS,)),
            pltpu.SemaphoreType.DMA((3, B, N_TILES)),
            pltpu.SemaphoreType.DMA,
            pltpu.SemaphoreType.REGULAR,
        ],
    )

    out, _ = pl.pallas_call(
        body,
        grid_spec=grid_spec,
        out_shape=(
            jax.ShapeDtypeStruct((B, S_CHUNK, N_OUT), jnp.float32),
            jax.ShapeDtypeStruct((3, B, N_TILES, S_CHUNK, NT), jnp.float32),
        ),
        compiler_params=pltpu.CompilerParams(
            dimension_semantics=("arbitrary", "arbitrary", "arbitrary"),
            collective_id=0,
            vmem_limit_bytes=100 * 1024 * 1024,
        ),
    )(idxs, A, Wo)
    return out


# device time: 2402319 ns/iter; 1.0420x vs baseline; 1.0420x over previous
import jax
import jax.numpy as jnp
from jax import lax
from jax.experimental import pallas as pl
from jax.experimental.pallas import tpu as pltpu

N_DEV = 4
B = 4
S_CHUNK = 512
K = 2048
N_OUT = 8192
NT = 1024
N_TILES = N_OUT // NT
TILES_PER_STEP = B * N_TILES
N_SLOTS = 4


def kernel(O, Wo):
    A = O.reshape(B, N_DEV * S_CHUNK, K)
    my = lax.axis_index("i")
    idxs = jnp.mod(my - 1 - jnp.arange(N_DEV, dtype=jnp.int32), N_DEV).astype(
        jnp.int32
    )

    def body(idx_ref, a_ref, w_ref, o_ref, recvbuf, send_slots, recv_vmem,
             send_sems, recv_sems, local_sem, exit_sem):
        h = pl.program_id(0)
        b = pl.program_id(1)
        n = pl.program_id(2)
        i = lax.axis_index("i")
        right = lax.rem(i + 1, N_DEV)
        left = lax.rem(i + N_DEV - 1, N_DEV)
        g = (h * B + b) * N_TILES + n
        slot = lax.rem(g, N_SLOTS)

        barrier = pltpu.get_barrier_semaphore()

        @pl.when(g == 0)
        def _entry_barrier():
            for nbr in (left, right):
                pl.semaphore_signal(
                    barrier, inc=1, device_id=(nbr,),
                    device_id_type=pl.DeviceIdType.MESH,
                )
            pl.semaphore_wait(barrier, 2)

        @pl.when((g >= N_SLOTS) & (g < 3 * TILES_PER_STEP + N_SLOTS))
        def _wait_old_send():
            d = pltpu.make_async_remote_copy(
                src_ref=send_slots.at[slot],
                dst_ref=recvbuf.at[0, 0, 0],
                send_sem=send_sems.at[slot],
                recv_sem=recv_sems.at[0, 0, 0],
                device_id=(right,),
                device_id_type=pl.DeviceIdType.MESH,
            )
            d.wait_send()

        @pl.when((g >= TILES_PER_STEP - 1) & (g < N_DEV * TILES_PER_STEP - 1))
        def _prefetch_recv():
            gn = g + 1 - TILES_PER_STEP
            hm = lax.div(gn, TILES_PER_STEP)
            bm = lax.rem(lax.div(gn, N_TILES), B)
            nm = lax.rem(gn, N_TILES)
            rd = pltpu.make_async_remote_copy(
                src_ref=send_slots.at[0],
                dst_ref=recvbuf.at[hm, bm, nm],
                send_sem=send_sems.at[0],
                recv_sem=recv_sems.at[hm, bm, nm],
                device_id=(left,),
                device_id_type=pl.DeviceIdType.MESH,
            )
            rd.wait_recv()
            pltpu.make_async_copy(
                recvbuf.at[hm, bm, nm],
                recv_vmem.at[lax.rem(g + 1, 2)],
                local_sem.at[lax.rem(g + 1, 2)],
            ).start()

        @pl.when(h > 0)
        def _wait_recv_copy():
            pltpu.make_async_copy(
                recvbuf.at[h - 1, b, n],
                recv_vmem.at[lax.rem(g, 2)],
                local_sem.at[lax.rem(g, 2)],
            ).wait()

        acc = jnp.dot(a_ref[0], w_ref[...], preferred_element_type=jnp.float32)

        @pl.when(h == 0)
        def _store_first():
            send_slots[slot] = acc

        @pl.when((h > 0) & (h < 3))
        def _store_mid():
            send_slots[slot] = acc + recv_vmem[lax.rem(g, 2)]

        @pl.when(h < 3)
        def _send():
            rdma = pltpu.make_async_remote_copy(
                src_ref=send_slots.at[slot],
                dst_ref=recvbuf.at[h, b, n],
                send_sem=send_sems.at[slot],
                recv_sem=recv_sems.at[h, b, n],
                device_id=(right,),
                device_id_type=pl.DeviceIdType.MESH,
            )
            rdma.start()

        @pl.when(h == 3)
        def _store():
            o_ref[0] = acc + recv_vmem[lax.rem(g, 2)]

        @pl.when(g == N_DEV * TILES_PER_STEP - 1)
        def _exit_barrier():
            for nbr in (left, right):
                pl.semaphore_signal(
                    exit_sem, inc=1, device_id=(nbr,),
                    device_id_type=pl.DeviceIdType.MESH,
                )
            pl.semaphore_wait(exit_sem, 2)

    grid_spec = pltpu.PrefetchScalarGridSpec(
        num_scalar_prefetch=1,
        grid=(N_DEV, B, N_TILES),
        in_specs=[
            pl.BlockSpec((1, S_CHUNK, K), lambda h, b, n, idx: (b, idx[h], 0)),
            pl.BlockSpec((K, NT), lambda h, b, n, idx: (0, n)),
        ],
        out_specs=[
            pl.BlockSpec((1, S_CHUNK, NT), lambda h, b, n, idx: (b, 0, n)),
            pl.BlockSpec(memory_space=pl.ANY),
        ],
        scratch_shapes=[
            pltpu.VMEM((N_SLOTS, S_CHUNK, NT), jnp.float32),
            pltpu.VMEM((2, S_CHUNK, NT), jnp.float32),
            pltpu.SemaphoreType.DMA((N_SLOTS,)),
            pltpu.SemaphoreType.DMA((3, B, N_TILES)),
            pltpu.SemaphoreType.DMA((2,)),
            pltpu.SemaphoreType.REGULAR,
        ],
    )

    out, _ = pl.pallas_call(
        body,
        grid_spec=grid_spec,
        out_shape=(
            jax.ShapeDtypeStruct((B, S_CHUNK, N_OUT), jnp.float32),
            jax.ShapeDtypeStruct((3, B, N_TILES, S_CHUNK, NT), jnp.float32),
        ),
        compiler_params=pltpu.CompilerParams(
            dimension_semantics=("arbitrary", "arbitrary", "arbitrary"),
            collective_id=0,
            vmem_limit_bytes=100 * 1024 * 1024,
        ),
    )(idxs, A, Wo)
    return out
